# baseline (speedup 1.0000x reference)
import functools

import jax
import jax.numpy as jnp
from jax import lax
from jax.experimental import pallas as pl
from jax.experimental.pallas import tpu as pltpu
from jax.experimental.pallas import tpu_sc as plsc

B, L, V, H = 1024, 50, 1000, 128
VP = 1024
NT = B * L
NW = 32
TW = NT // NW
CH = 40
NCHUNK = TW // CH
LANES = 16


def _tables_body(a_ref, wo_ref, wt_ref, bt_ref, bo_ref, p_ref, r_ref):
    wo = wo_ref[...]
    c = jnp.dot(bt_ref[...], wo, preferred_element_type=jnp.float32) + bo_ref[...]
    r_ref[...] = jnp.dot(wt_ref[...], wo, preferred_element_type=jnp.float32)
    p_ref[...] = jnp.dot(a_ref[...], wo, preferred_element_type=jnp.float32) + c


def _build_tables(emb_p, wo_p, wt, bt2, bo_p):
    grid = (VP // 128,)
    return pl.pallas_call(
        _tables_body,
        grid=grid,
        in_specs=[
            pl.BlockSpec((128, H), lambda i: (i, 0)),
            pl.BlockSpec((H, VP), lambda i: (0, 0)),
            pl.BlockSpec((1, H), lambda i: (0, 0)),
            pl.BlockSpec((1, H), lambda i: (0, 0)),
            pl.BlockSpec((1, VP), lambda i: (0, 0)),
        ],
        out_specs=[
            pl.BlockSpec((128, VP), lambda i: (i, 0)),
            pl.BlockSpec((1, VP), lambda i: (0, 0)),
        ],
        out_shape=[
            jax.ShapeDtypeStruct((VP, VP), jnp.float32),
            jax.ShapeDtypeStruct((1, VP), jnp.float32),
        ],
    )(emb_p, wo_p, wt, bt2, bo_p)


def _make_sc_gather():
    mesh = plsc.VectorSubcoreMesh(core_axis_name="c", subcore_axis_name="s")

    @functools.partial(
        pl.kernel,
        mesh=mesh,
        out_type=jax.ShapeDtypeStruct((NT, V), jnp.float32),
        scratch_types=[
            pltpu.VMEM((TW,), jnp.int32),
            pltpu.VMEM((TW // 8, 8 * LANES), jnp.float32),
            pltpu.VMEM((1, VP), jnp.float32),
            pltpu.VMEM((CH, VP), jnp.float32),
            pltpu.VMEM((CH, V), jnp.float32),
            pltpu.SemaphoreType.DMA,
        ],
    )
    def sc_gather(p_hbm, r_hbm, idx_hbm, tt_hbm, out_hbm,
                  idx_v, tt_v, r_v, rows_v, rows_c, sem):
        wid = lax.axis_index("s") * 2 + lax.axis_index("c")
        base = wid * TW
        pltpu.sync_copy(idx_hbm.at[wid], idx_v)
        pltpu.sync_copy(tt_hbm.at[wid], tt_v)
        pltpu.sync_copy(r_hbm, r_v)
        TAIL = V - LANES

        def chunk(k, carry):
            tok = base + k * CH
            pltpu.async_copy(p_hbm.at[idx_v.at[pl.ds(k * CH, CH)]],
                             rows_v, sem).wait()

            def row(j, c2):
                jj = k * CH + j
                tj = tt_v[jj >> 3, pl.ds((jj & 7) * LANES, LANES)]
                sl = pl.ds(TAIL, LANES)
                rows_c[j, sl] = rows_v[j, sl] + tj * r_v[0, sl]
                for cc in range(V // LANES):
                    sl = pl.ds(cc * LANES, LANES)
                    rows_c[j, sl] = rows_v[j, sl] + tj * r_v[0, sl]
                return c2

            lax.fori_loop(0, CH, row, 0)
            pltpu.sync_copy(rows_c, out_hbm.at[pl.ds(tok, CH)])
            return carry

        lax.fori_loop(0, NCHUNK, chunk, 0)

    return sc_gather


_sc_gather = _make_sc_gather()


def kernel(x, t, emb, Wt, bt, Wo, bo):
    xflat = x.reshape(NW, TW).astype(jnp.int32)
    ttok = jnp.broadcast_to(
        jnp.repeat(t.astype(jnp.float32), L)[:, None], (NT, LANES)
    ).reshape(NW, TW // 8, 8 * LANES)
    emb_p = jnp.pad(emb, ((0, VP - V), (0, 0)))
    wo_p = jnp.pad(Wo, ((0, 0), (0, VP - V)))
    bo_p = jnp.pad(bo, (0, VP - V)).reshape(1, VP)
    bt2 = bt.reshape(1, H)
    P, r = _build_tables(emb_p, wo_p, Wt, bt2, bo_p)
    out = _sc_gather(P, r, xflat, ttok)
    return out.reshape(B, L, V)


# speedup vs baseline: 1.1725x; 1.1725x over previous
import functools

import jax
import jax.numpy as jnp
from jax import lax
from jax.experimental import pallas as pl
from jax.experimental.pallas import tpu as pltpu
from jax.experimental.pallas import tpu_sc as plsc

B, L, V, H = 1024, 50, 1000, 128
VP = 1024
NT = B * L
NW = 32
TW = NT // NW
CH = 16
NCHUNK = TW // CH
NBUF = 2
LANES = 16


def _tables_body(a_ref, wo_ref, wt_ref, bt_ref, bo_ref, p_ref, r_ref):
    wo = wo_ref[...]
    c = jnp.dot(bt_ref[...], wo, preferred_element_type=jnp.float32) + bo_ref[...]
    r_ref[...] = jnp.dot(wt_ref[...], wo, preferred_element_type=jnp.float32)
    p_ref[...] = jnp.dot(a_ref[...], wo, preferred_element_type=jnp.float32) + c


def _build_tables(emb_p, wo_p, wt, bt2, bo_p):
    grid = (VP // 128,)
    return pl.pallas_call(
        _tables_body,
        grid=grid,
        in_specs=[
            pl.BlockSpec((128, H), lambda i: (i, 0)),
            pl.BlockSpec((H, VP), lambda i: (0, 0)),
            pl.BlockSpec((1, H), lambda i: (0, 0)),
            pl.BlockSpec((1, H), lambda i: (0, 0)),
            pl.BlockSpec((1, VP), lambda i: (0, 0)),
        ],
        out_specs=[
            pl.BlockSpec((128, VP), lambda i: (i, 0)),
            pl.BlockSpec((1, VP), lambda i: (0, 0)),
        ],
        out_shape=[
            jax.ShapeDtypeStruct((VP, VP), jnp.float32),
            jax.ShapeDtypeStruct((1, VP), jnp.float32),
        ],
    )(emb_p, wo_p, wt, bt2, bo_p)


def _make_sc_gather():
    mesh = plsc.VectorSubcoreMesh(core_axis_name="c", subcore_axis_name="s")

    @functools.partial(
        pl.kernel,
        mesh=mesh,
        out_type=jax.ShapeDtypeStruct((NT, V), jnp.float32),
        scratch_types=[
            pltpu.VMEM((TW,), jnp.int32),
            pltpu.VMEM((TW // 8, 8 * LANES), jnp.float32),
            pltpu.VMEM((1, VP), jnp.float32),
            pltpu.VMEM((NBUF, CH, VP), jnp.float32),
            pltpu.VMEM((NBUF, CH, V), jnp.float32),
            pltpu.SemaphoreType.DMA,
            pltpu.SemaphoreType.DMA,
            pltpu.SemaphoreType.DMA,
            pltpu.SemaphoreType.DMA,
        ],
    )
    def sc_gather(p_hbm, r_hbm, idx_hbm, tt_hbm, out_hbm,
                  idx_v, tt_v, r_v, rows_v, rows_c,
                  gsem0, gsem1, wsem0, wsem1):
        gsem = (gsem0, gsem1)
        wsem = (wsem0, wsem1)
        wid = lax.axis_index("s") * 2 + lax.axis_index("c")
        base = wid * TW
        pltpu.sync_copy(idx_hbm.at[wid], idx_v)
        pltpu.sync_copy(tt_hbm.at[wid], tt_v)
        pltpu.sync_copy(r_hbm, r_v)
        TAIL = V - LANES

        def gather_start(k, b):
            pltpu.async_copy(p_hbm.at[idx_v.at[pl.ds(k * CH, CH)]],
                             rows_v.at[b], gsem[b])

        def gather_wait(k, b):
            pltpu.make_async_copy(p_hbm.at[idx_v.at[pl.ds(k * CH, CH)]],
                                  rows_v.at[b], gsem[b]).wait()

        def write_start(k, b):
            pltpu.async_copy(rows_c.at[b],
                             out_hbm.at[pl.ds(base + k * CH, CH)], wsem[b])

        def write_wait(k, b):
            pltpu.make_async_copy(rows_c.at[b],
                                  out_hbm.at[pl.ds(base + k * CH, CH)],
                                  wsem[b]).wait()

        for b in range(NBUF):
            gather_start(b, b)

        def pair(p_, carry):
            for b in range(NBUF):
                k = p_ * NBUF + b
                gather_wait(k, b)

                @pl.when(k >= NBUF)
                def _():
                    write_wait(k - NBUF, b)

                rv = rows_v.at[b]
                rc = rows_c.at[b]

                def row(j, c2):
                    jj = k * CH + j
                    tj = tt_v[jj >> 3, pl.ds((jj & 7) * LANES, LANES)]
                    sl = pl.ds(TAIL, LANES)
                    rc[j, sl] = rv[j, sl] + tj * r_v[0, sl]
                    for cc in range(V // LANES):
                        sl = pl.ds(cc * LANES, LANES)
                        rc[j, sl] = rv[j, sl] + tj * r_v[0, sl]
                    return c2

                lax.fori_loop(0, CH, row, 0)
                write_start(k, b)

                @pl.when(k + NBUF < NCHUNK)
                def _():
                    gather_start(k + NBUF, b)
            return carry

        lax.fori_loop(0, NCHUNK // NBUF, pair, 0)
        for b in range(NBUF):
            write_wait(NCHUNK - NBUF + b, b)

    return sc_gather


_sc_gather = _make_sc_gather()


def kernel(x, t, emb, Wt, bt, Wo, bo):
    xflat = x.reshape(NW, TW).astype(jnp.int32)
    ttok = jnp.broadcast_to(
        jnp.repeat(t.astype(jnp.float32), L)[:, None], (NT, LANES)
    ).reshape(NW, TW // 8, 8 * LANES)
    emb_p = jnp.pad(emb, ((0, VP - V), (0, 0)))
    wo_p = jnp.pad(Wo, ((0, 0), (0, VP - V)))
    bo_p = jnp.pad(bo, (0, VP - V)).reshape(1, VP)
    bt2 = bt.reshape(1, H)
    P, r = _build_tables(emb_p, wo_p, Wt, bt2, bo_p)
    out = _sc_gather(P, r, xflat, ttok)
    return out.reshape(B, L, V)


# speedup vs baseline: 1.2413x; 1.0586x over previous
import functools

import jax
import jax.numpy as jnp
from jax import lax
from jax.experimental import pallas as pl
from jax.experimental.pallas import tpu as pltpu
from jax.experimental.pallas import tpu_sc as plsc

B, L, V, H = 1024, 50, 1000, 128
VP = 1024
NW = 32
BPW = B // NW
CLS = (16, 16, 18)
GCLS = (16, 16, 24)
L0S = (0, 16, 32)
SEG = 3
NCHUNK = BPW * SEG
CMAX = 24
CCMAX = 18
NBUF = 2
LANES = 16


def _tables_body(a_ref, wo_ref, wt_ref, bt_ref, bo_ref, p_ref, r_ref):
    wo = wo_ref[...]
    c = jnp.dot(bt_ref[...], wo, preferred_element_type=jnp.float32) + bo_ref[...]
    r_ref[...] = jnp.dot(wt_ref[...], wo, preferred_element_type=jnp.float32)
    p_ref[...] = jnp.dot(a_ref[...], wo, preferred_element_type=jnp.float32) + c


def _build_tables(emb_p, wo_p, wt, bt2, bo_p):
    grid = (VP // 128,)
    return pl.pallas_call(
        _tables_body,
        grid=grid,
        in_specs=[
            pl.BlockSpec((128, H), lambda i: (i, 0)),
            pl.BlockSpec((H, VP), lambda i: (0, 0)),
            pl.BlockSpec((1, H), lambda i: (0, 0)),
            pl.BlockSpec((1, H), lambda i: (0, 0)),
            pl.BlockSpec((1, VP), lambda i: (0, 0)),
        ],
        out_specs=[
            pl.BlockSpec((128, VP), lambda i: (i, 0)),
            pl.BlockSpec((1, VP), lambda i: (0, 0)),
        ],
        out_shape=[
            jax.ShapeDtypeStruct((VP, VP), jnp.float32),
            jax.ShapeDtypeStruct((1, VP), jnp.float32),
        ],
    )(emb_p, wo_p, wt, bt2, bo_p)


def _make_sc_gather():
    mesh = plsc.VectorSubcoreMesh(core_axis_name="c", subcore_axis_name="s")

    @functools.partial(
        pl.kernel,
        mesh=mesh,
        out_type=jax.ShapeDtypeStruct((B, L, V), jnp.float32),
        scratch_types=[
            pltpu.VMEM((NCHUNK, CMAX), jnp.int32),
            pltpu.VMEM((BPW, LANES), jnp.float32),
            pltpu.VMEM((1, VP), jnp.float32),
            pltpu.VMEM((NBUF, CMAX, VP), jnp.float32),
            pltpu.VMEM((NBUF, CCMAX, V), jnp.float32),
            pltpu.SemaphoreType.DMA,
            pltpu.SemaphoreType.DMA,
            pltpu.SemaphoreType.DMA,
            pltpu.SemaphoreType.DMA,
        ],
    )
    def sc_gather(p_hbm, r_hbm, idx_hbm, tt_hbm, out_hbm,
                  idx_v, tt_v, r_v, rows_v, rows_c,
                  gsem0, gsem1, wsem0, wsem1):
        gsem = (gsem0, gsem1)
        wsem = (wsem0, wsem1)
        wid = lax.axis_index("s") * 2 + lax.axis_index("c")
        pltpu.sync_copy(idx_hbm.at[wid], idx_v)
        pltpu.sync_copy(tt_hbm.at[wid], tt_v)
        pltpu.sync_copy(r_hbm, r_v)
        TAIL = V - LANES
        TSRC = VP - LANES

        def gstart(qv, p, b):
            cl = GCLS[p]
            pltpu.async_copy(
                p_hbm.at[idx_v.at[SEG * qv + p, pl.ds(0, cl)]],
                rows_v.at[b, pl.ds(0, cl)], gsem[b])

        def gwait(qv, p, b):
            cl = GCLS[p]
            pltpu.make_async_copy(
                p_hbm.at[idx_v.at[SEG * qv + p, pl.ds(0, cl)]],
                rows_v.at[b, pl.ds(0, cl)], gsem[b]).wait()

        def wstart(qv, p, b):
            cl = CLS[p]
            pltpu.async_copy(
                rows_c.at[b, pl.ds(0, cl)],
                out_hbm.at[wid * BPW + qv, pl.ds(L0S[p], cl)], wsem[b])

        def wwait(qv, p, b):
            cl = CLS[p]
            pltpu.make_async_copy(
                rows_c.at[b, pl.ds(0, cl)],
                out_hbm.at[wid * BPW + qv, pl.ds(L0S[p], cl)],
                wsem[b]).wait()

        def compute(qv, p, b):
            tj = tt_v[qv, :]

            def row(j, c2):
                sl = pl.ds(TSRC, LANES)
                rows_c[b, j, pl.ds(TAIL, LANES)] = (
                    rows_v[b, j, sl] + tj * r_v[0, sl])
                for cc in range(V // LANES):
                    sl = pl.ds(cc * LANES, LANES)
                    rows_c[b, j, sl] = rows_v[b, j, sl] + tj * r_v[0, sl]
                return c2

            lax.fori_loop(0, CLS[p], row, 0)

        gstart(0, 0, 0)
        gstart(0, 1, 1)

        def pair_body(pr, carry):
            for i in range(2 * SEG):
                p = i % SEG
                b = i % NBUF
                qv = 2 * pr + i // SEG
                k = 2 * SEG * pr + i
                gwait(qv, p, b)

                pm = (i - NBUF) % SEG
                qm = 2 * pr + (i - NBUF) // SEG

                @pl.when(k >= NBUF)
                def _(qm=qm, pm=pm, b=b):
                    wwait(qm, pm, b)

                compute(qv, p, b)
                wstart(qv, p, b)

                pp = (i + NBUF) % SEG
                qp = 2 * pr + (i + NBUF) // SEG

                @pl.when(k + NBUF < NCHUNK)
                def _(qp=qp, pp=pp, b=b):
                    gstart(qp, pp, b)
            return carry

        lax.fori_loop(0, BPW // 2, pair_body, 0)
        wwait(BPW - 2 + 4 // SEG, 4 % SEG, 0)
        wwait(BPW - 2 + 5 // SEG, 5 % SEG, 1)

    return sc_gather


_sc_gather = _make_sc_gather()


def kernel(x, t, emb, Wt, bt, Wo, bo):
    xw = x.reshape(NW, BPW, L).astype(jnp.int32)
    parts = [
        jnp.pad(xw[:, :, L0S[p]:L0S[p] + CLS[p]],
                ((0, 0), (0, 0), (0, CMAX - CLS[p])))
        for p in range(SEG)
    ]
    idx = jnp.stack(parts, axis=2).reshape(NW, NCHUNK, CMAX)
    del xw
    tt = jnp.broadcast_to(
        t.astype(jnp.float32).reshape(NW, BPW, 1), (NW, BPW, LANES))
    emb_p = jnp.pad(emb, ((0, VP - V), (0, 0)))
    wo_p = jnp.concatenate(
        [Wo, jnp.zeros((H, VP - V - LANES), jnp.float32),
         Wo[:, V - LANES:]], axis=1)
    bo_p = jnp.concatenate(
        [bo, jnp.zeros((VP - V - LANES,), jnp.float32),
         bo[V - LANES:]]).reshape(1, VP)
    bt2 = bt.reshape(1, H)
    P, r = _build_tables(emb_p, wo_p, Wt, bt2, bo_p)
    return _sc_gather(P, r, idx, tt)


# speedup vs baseline: 1.2750x; 1.0271x over previous
import functools

import jax
import jax.numpy as jnp
from jax import lax
from jax.experimental import pallas as pl
from jax.experimental.pallas import tpu as pltpu
from jax.experimental.pallas import tpu_sc as plsc

B, L, V, H = 1024, 50, 1000, 128
VP = 1024
NW = 32
BPW = B // NW
CLS = (16, 16, 18)
GCLS = (16, 16, 24)
L0S = (0, 16, 32)
SEG = 3
NCHUNK = BPW * SEG
CMAX = 24
CCMAX = 18
NBUF = 2
LANES = 16


def _tables_body(a_ref, wo_ref, wt_ref, bt_ref, bo_ref, p_ref, r_ref):
    wo = wo_ref[...]
    c = jnp.dot(bt_ref[...], wo, preferred_element_type=jnp.float32) + bo_ref[...]
    r_ref[...] = jnp.dot(wt_ref[...], wo, preferred_element_type=jnp.float32)
    p_ref[...] = jnp.dot(a_ref[...], wo, preferred_element_type=jnp.float32) + c


def _build_tables(emb_p, wo_p, wt, bt2, bo_p):
    grid = (VP // 128,)
    return pl.pallas_call(
        _tables_body,
        grid=grid,
        in_specs=[
            pl.BlockSpec((128, H), lambda i: (i, 0)),
            pl.BlockSpec((H, VP), lambda i: (0, 0)),
            pl.BlockSpec((1, H), lambda i: (0, 0)),
            pl.BlockSpec((1, H), lambda i: (0, 0)),
            pl.BlockSpec((1, VP), lambda i: (0, 0)),
        ],
        out_specs=[
            pl.BlockSpec((128, VP), lambda i: (i, 0)),
            pl.BlockSpec((1, VP), lambda i: (0, 0)),
        ],
        out_shape=[
            jax.ShapeDtypeStruct((VP, VP), jnp.float32),
            jax.ShapeDtypeStruct((1, VP), jnp.float32),
        ],
    )(emb_p, wo_p, wt, bt2, bo_p)


def _make_sc_gather():
    mesh = plsc.VectorSubcoreMesh(core_axis_name="c", subcore_axis_name="s")

    @functools.partial(
        pl.kernel,
        mesh=mesh,
        out_type=jax.ShapeDtypeStruct((B, L, V), jnp.float32),
        scratch_types=[
            pltpu.VMEM((NCHUNK, CMAX), jnp.int32),
            pltpu.VMEM((BPW, LANES), jnp.float32),
            pltpu.VMEM((1, VP), jnp.float32),
            pltpu.VMEM((NBUF, CMAX, VP), jnp.float32),
            pltpu.VMEM((NBUF, CCMAX, V), jnp.float32),
            pltpu.SemaphoreType.DMA,
            pltpu.SemaphoreType.DMA,
            pltpu.SemaphoreType.DMA,
            pltpu.SemaphoreType.DMA,
        ],
    )
    def sc_gather(p_hbm, r_hbm, idx_hbm, tt_hbm, out_hbm,
                  idx_v, tt_v, r_v, rows_v, rows_c,
                  gsem0, gsem1, wsem0, wsem1):
        gsem = (gsem0, gsem1)
        wsem = (wsem0, wsem1)
        wid = lax.axis_index("s") * 2 + lax.axis_index("c")
        pltpu.sync_copy(idx_hbm.at[wid], idx_v)
        pltpu.sync_copy(tt_hbm.at[wid], tt_v)
        pltpu.sync_copy(r_hbm, r_v)
        TAIL = V - LANES
        TSRC = VP - LANES

        def gstart(qv, p, b):
            cl = GCLS[p]
            pltpu.async_copy(
                p_hbm.at[idx_v.at[SEG * qv + p, pl.ds(0, cl)]],
                rows_v.at[b, pl.ds(0, cl)], gsem[b])

        def gwait(qv, p, b):
            cl = GCLS[p]
            pltpu.make_async_copy(
                p_hbm.at[idx_v.at[SEG * qv + p, pl.ds(0, cl)]],
                rows_v.at[b, pl.ds(0, cl)], gsem[b]).wait()

        def wstart(qv, p, b):
            cl = CLS[p]
            pltpu.async_copy(
                rows_c.at[b, pl.ds(0, cl)],
                out_hbm.at[wid * BPW + qv, pl.ds(L0S[p], cl)], wsem[b])

        def wwait(qv, p, b):
            cl = CLS[p]
            pltpu.make_async_copy(
                rows_c.at[b, pl.ds(0, cl)],
                out_hbm.at[wid * BPW + qv, pl.ds(L0S[p], cl)],
                wsem[b]).wait()

        def compute(qv, p, b):
            tj = tt_v[qv, :]

            @plsc.parallel_loop(0, CLS[p], unroll=1)
            def row(j):
                sl = pl.ds(TSRC, LANES)
                rows_c[b, j, pl.ds(TAIL, LANES)] = (
                    rows_v[b, j, sl] + tj * r_v[0, sl])
                for cc in range(V // LANES):
                    sl = pl.ds(cc * LANES, LANES)
                    rows_c[b, j, sl] = rows_v[b, j, sl] + tj * r_v[0, sl]

        gstart(0, 0, 0)
        gstart(0, 1, 1)

        def pair_body(pr, carry):
            for i in range(2 * SEG):
                p = i % SEG
                b = i % NBUF
                qv = 2 * pr + i // SEG
                k = 2 * SEG * pr + i
                gwait(qv, p, b)

                pm = (i - NBUF) % SEG
                qm = 2 * pr + (i - NBUF) // SEG

                @pl.when(k >= NBUF)
                def _(qm=qm, pm=pm, b=b):
                    wwait(qm, pm, b)

                compute(qv, p, b)
                wstart(qv, p, b)

                pp = (i + NBUF) % SEG
                qp = 2 * pr + (i + NBUF) // SEG

                @pl.when(k + NBUF < NCHUNK)
                def _(qp=qp, pp=pp, b=b):
                    gstart(qp, pp, b)
            return carry

        lax.fori_loop(0, BPW // 2, pair_body, 0)
        wwait(BPW - 2 + 4 // SEG, 4 % SEG, 0)
        wwait(BPW - 2 + 5 // SEG, 5 % SEG, 1)

    return sc_gather


_sc_gather = _make_sc_gather()


def kernel(x, t, emb, Wt, bt, Wo, bo):
    xw = x.reshape(NW, BPW, L).astype(jnp.int32)
    parts = [
        jnp.pad(xw[:, :, L0S[p]:L0S[p] + CLS[p]],
                ((0, 0), (0, 0), (0, CMAX - CLS[p])))
        for p in range(SEG)
    ]
    idx = jnp.stack(parts, axis=2).reshape(NW, NCHUNK, CMAX)
    del xw
    tt = jnp.broadcast_to(
        t.astype(jnp.float32).reshape(NW, BPW, 1), (NW, BPW, LANES))
    emb_p = jnp.pad(emb, ((0, VP - V), (0, 0)))
    wo_p = jnp.concatenate(
        [Wo, jnp.zeros((H, VP - V - LANES), jnp.float32),
         Wo[:, V - LANES:]], axis=1)
    bo_p = jnp.concatenate(
        [bo, jnp.zeros((VP - V - LANES,), jnp.float32),
         bo[V - LANES:]]).reshape(1, VP)
    bt2 = bt.reshape(1, H)
    P, r = _build_tables(emb_p, wo_p, Wt, bt2, bo_p)
    return _sc_gather(P, r, idx, tt)


# speedup vs baseline: 1.3375x; 1.0491x over previous
import functools

import jax
import jax.numpy as jnp
from jax import lax
from jax.experimental import pallas as pl
from jax.experimental.pallas import tpu as pltpu
from jax.experimental.pallas import tpu_sc as plsc

B, L, V, H = 1024, 50, 1000, 128
VP = 1024
NW = 32
BPW = B // NW
FSEG = 6
CH = 8
TL = L - FSEG * CH
SEG = FSEG + 1
NFULL = BPW * FSEG
NBUF = 6
LANES = 16


def _tables_body(a_ref, wo_ref, wt_ref, bt_ref, bo_ref, p_ref, r_ref):
    wo = wo_ref[...]
    c = jnp.dot(bt_ref[...], wo, preferred_element_type=jnp.float32) + bo_ref[...]
    r_ref[...] = jnp.dot(wt_ref[...], wo, preferred_element_type=jnp.float32)
    p_ref[...] = jnp.dot(a_ref[...], wo, preferred_element_type=jnp.float32) + c


def _build_tables(emb_p, wo_p, wt, bt2, bo_p):
    grid = (VP // 128,)
    return pl.pallas_call(
        _tables_body,
        grid=grid,
        in_specs=[
            pl.BlockSpec((128, H), lambda i: (i, 0)),
            pl.BlockSpec((H, VP), lambda i: (0, 0)),
            pl.BlockSpec((1, H), lambda i: (0, 0)),
            pl.BlockSpec((1, H), lambda i: (0, 0)),
            pl.BlockSpec((1, VP), lambda i: (0, 0)),
        ],
        out_specs=[
            pl.BlockSpec((128, VP), lambda i: (i, 0)),
            pl.BlockSpec((1, VP), lambda i: (0, 0)),
        ],
        out_shape=[
            jax.ShapeDtypeStruct((VP, VP), jnp.float32),
            jax.ShapeDtypeStruct((1, VP), jnp.float32),
        ],
    )(emb_p, wo_p, wt, bt2, bo_p)


def _make_sc_gather():
    mesh = plsc.VectorSubcoreMesh(core_axis_name="c", subcore_axis_name="s")

    @functools.partial(
        pl.kernel,
        mesh=mesh,
        out_type=jax.ShapeDtypeStruct((B, L, V), jnp.float32),
        scratch_types=[
            pltpu.VMEM((SEG * BPW * CH,), jnp.int32),
            pltpu.VMEM((BPW // 8, 8 * LANES), jnp.float32),
            pltpu.VMEM((1, VP), jnp.float32),
            pltpu.VMEM((NBUF, CH, VP), jnp.float32),
            pltpu.VMEM((NBUF, CH, V), jnp.float32),
            pltpu.VMEM((CH, VP), jnp.float32),
            pltpu.VMEM((TL, V), jnp.float32),
            pltpu.SemaphoreType.DMA((NBUF,)),
            pltpu.SemaphoreType.DMA((NBUF,)),
            pltpu.SemaphoreType.DMA,
            pltpu.SemaphoreType.DMA,
        ],
    )
    def sc_gather(p_hbm, r_hbm, idx_hbm, tt_hbm, out_hbm,
                  idx_v, tt_v, r_v, rows_v, rows_c, rows_vt, rows_ct,
                  gsems, wsems, gsemt, wsemt):
        gsem = tuple(gsems.at[i] for i in range(NBUF))
        wsem = tuple(wsems.at[i] for i in range(NBUF))
        sid = lax.axis_index("s")
        wid = sid * 2 + lax.axis_index("c")
        pltpu.sync_copy(idx_hbm.at[wid], idx_v)
        pltpu.sync_copy(tt_hbm.at[wid], tt_v)
        pltpu.sync_copy(r_hbm, r_v)
        TAIL = V - LANES
        TSRC = VP - LANES

        def islice(k):
            return idx_v.at[pl.ds(k * CH, CH)]

        def tsplat(qv):
            return tt_v[qv >> 3, pl.ds((qv & 7) * LANES, LANES)]

        def gstart(qv, p, b):
            pltpu.async_copy(p_hbm.at[islice(SEG * qv + p)],
                             rows_v.at[b], gsem[b])

        def gwait(qv, p, b):
            pltpu.make_async_copy(p_hbm.at[islice(SEG * qv + p)],
                                  rows_v.at[b], gsem[b]).wait()

        def wstart(qv, p, b):
            pltpu.async_copy(
                rows_c.at[b],
                out_hbm.at[wid * BPW + qv, pl.ds(p * CH, CH)], wsem[b])

        def wwait(qv, p, b):
            pltpu.make_async_copy(
                rows_c.at[b],
                out_hbm.at[wid * BPW + qv, pl.ds(p * CH, CH)],
                wsem[b]).wait()

        def add_rows(dst, src, tj, nrows):
            @plsc.parallel_loop(0, nrows, unroll=1)
            def row(j):
                sl = pl.ds(TSRC, LANES)
                dst[j, pl.ds(TAIL, LANES)] = src[j, sl] + tj * r_v[0, sl]
                for cc in range(V // LANES):
                    sl = pl.ds(cc * LANES, LANES)
                    dst[j, sl] = src[j, sl] + tj * r_v[0, sl]

        for p in range(FSEG):
            gstart(0, p, p)

        def batch_body(qv, carry):
            bq = wid * BPW + qv
            tj = tsplat(qv)

            @pl.when(qv >= 1)
            def _():
                pltpu.make_async_copy(
                    rows_ct, out_hbm.at[bq - 1, pl.ds(FSEG * CH, TL)],
                    wsemt).wait()

            pltpu.async_copy(p_hbm.at[islice(SEG * qv + FSEG)],
                             rows_vt, gsemt)

            for p in range(FSEG):
                b = p
                gwait(qv, p, b)

                @pl.when(qv >= 1)
                def _(qv=qv, p=p, b=b):
                    wwait(qv - 1, p, b)

                add_rows(rows_c.at[b], rows_v.at[b], tj, CH)
                wstart(qv, p, b)

                @pl.when(qv + 1 < BPW)
                def _(qv=qv, p=p, b=b):
                    gstart(qv + 1, p, b)

            pltpu.make_async_copy(p_hbm.at[islice(SEG * qv + FSEG)],
                                  rows_vt, gsemt).wait()
            add_rows(rows_ct, rows_vt, tj, TL)
            pltpu.async_copy(rows_ct,
                             out_hbm.at[bq, pl.ds(FSEG * CH, TL)], wsemt)
            return carry

        lax.fori_loop(0, BPW, batch_body, 0)
        for p in range(FSEG):
            wwait(BPW - 1, p, p)
        pltpu.make_async_copy(
            rows_ct,
            out_hbm.at[wid * BPW + BPW - 1, pl.ds(FSEG * CH, TL)],
            wsemt).wait()

    return sc_gather


_sc_gather = _make_sc_gather()


def kernel(x, t, emb, Wt, bt, Wo, bo):
    xw = x.reshape(NW, BPW, L).astype(jnp.int32)
    tail_idx = jnp.pad(xw[:, :, FSEG * CH:], ((0, 0), (0, 0), (0, CH - TL)))
    idx = jnp.concatenate([xw[:, :, :FSEG * CH], tail_idx], axis=2)
    idx = idx.reshape(NW, SEG * BPW * CH)
    tt = jnp.broadcast_to(
        t.astype(jnp.float32).reshape(NW, BPW, 1), (NW, BPW, LANES)
    ).reshape(NW, BPW // 8, 8 * LANES)
    emb_p = jnp.pad(emb, ((0, VP - V), (0, 0)))
    wo_p = jnp.concatenate(
        [Wo, jnp.zeros((H, VP - V - LANES), jnp.float32),
         Wo[:, V - LANES:]], axis=1)
    bo_p = jnp.concatenate(
        [bo, jnp.zeros((VP - V - LANES,), jnp.float32),
         bo[V - LANES:]]).reshape(1, VP)
    bt2 = bt.reshape(1, H)
    P, r = _build_tables(emb_p, wo_p, Wt, bt2, bo_p)
    return _sc_gather(P, r, idx, tt)
